# Initial kernel scaffold; baseline (speedup 1.0000x reference)
import jax
import jax.numpy as jnp
from jax.experimental import pallas as pl


def kernel(x, layer, keys, prompts):
    raise NotImplementedError("write your pallas kernel here")


# speedup vs baseline: 1.5904x; 1.5904x over previous
import functools

import jax
import jax.numpy as jnp
from jax import lax
from jax.experimental import pallas as pl
from jax.experimental.pallas import tpu as pltpu
from jax.experimental.pallas import tpu_sc as plsc

B = 4096
NUM_LAYERS = 12
POOL_SIZE = 1000
LEN_PROMPTS = 4
DIM = 768
TOPK = 9
ROW = LEN_PROMPTS * DIM
NIDX = B * TOPK
IDX_PAD = 16

BLK_B = 512
GRID = B // BLK_B

NC = 2
NS = 16
NW = NC * NS
BPW = NIDX // NW
CH = 16
NCHUNK = BPW // CH


def _lane_reduce(v):
    r = v[:, 0:8]
    for j in range(1, 16):
        r = r + v[:, 8 * j:8 * j + 8]
    r = r[:, 0:4] + r[:, 4:8]
    r = r[:, 0:2] + r[:, 2:4]
    return r[:, 0:1] + r[:, 1:2]


def _norm_body(layer_ref, x_ref, keys_ref, xn_ref, kn_ref, rm_ref, kbar_ref):
    @pl.when(pl.program_id(0) == 0)
    def _init():
        k = keys_ref[0]
        sq = k * k
        acc = sq[:, 0:128]
        for i in range(1, 6):
            acc = acc + sq[:, 128 * i:128 * (i + 1)]
        n2 = _lane_reduce(acc)
        kn = k / jnp.maximum(jnp.sqrt(n2), 1e-12)
        kn_ref[...] = kn
        kbar_ref[...] = jnp.sum(kn, axis=0, keepdims=True) / jnp.float32(POOL_SIZE)

    x = x_ref[...]
    sq = x * x
    p = _lane_reduce(sq[:, 0:128] + sq[:, 128:256])
    p = p + _lane_reduce(sq[:, 256:384] + sq[:, 384:512])
    p = p + _lane_reduce(sq[:, 512:640] + sq[:, 640:768])
    xn = x / jnp.maximum(jnp.sqrt(p), 1e-12)
    xn_ref[...] = xn
    rm_ref[...] = jnp.sum(xn * kbar_ref[0][None, :], axis=1)


def _normalize(layer_arr, x, keys):
    return pl.pallas_call(
        _norm_body,
        grid_spec=pltpu.PrefetchScalarGridSpec(
            num_scalar_prefetch=1,
            grid=(GRID,),
            in_specs=[
                pl.BlockSpec((BLK_B, DIM), lambda i, L: (i, 0)),
                pl.BlockSpec((1, POOL_SIZE, DIM), lambda i, L: (L[0], 0, 0)),
            ],
            out_specs=[
                pl.BlockSpec((BLK_B, DIM), lambda i, L: (i, 0)),
                pl.BlockSpec((POOL_SIZE, DIM), lambda i, L: (0, 0)),
                pl.BlockSpec((BLK_B,), lambda i, L: (i,)),
            ],
            scratch_shapes=[
                pltpu.VMEM((1, DIM), jnp.float32),
            ],
        ),
        out_shape=[
            jax.ShapeDtypeStruct((B, DIM), jnp.float32),
            jax.ShapeDtypeStruct((POOL_SIZE, DIM), jnp.float32),
            jax.ShapeDtypeStruct((B,), jnp.float32),
        ],
    )(layer_arr, x, keys)


def _topk_body(layer_ref, xb_ref, kb_ref, idx_ref):
    sim = lax.dot_general(xb_ref[...], kb_ref[...], (((1,), (1,)), ((), ())),
                          preferred_element_type=jnp.float32)
    it = lax.broadcasted_iota(jnp.int32, sim.shape, 1)
    off = layer_ref[0] * POOL_SIZE
    s = sim
    cols = []
    for _ in range(TOPK):
        m = jnp.max(s, axis=1, keepdims=True)
        j = jnp.min(jnp.where(s == m, it, jnp.int32(2**30)), axis=1, keepdims=True)
        cols.append(j + off)
        s = jnp.where(it == j, -jnp.inf, s)
    for _ in range(IDX_PAD - TOPK):
        cols.append(cols[-1])
    idx_ref[...] = jnp.concatenate(cols, axis=1)


def _topk(layer_arr, xb, kb):
    return pl.pallas_call(
        _topk_body,
        grid_spec=pltpu.PrefetchScalarGridSpec(
            num_scalar_prefetch=1,
            grid=(GRID,),
            in_specs=[
                pl.BlockSpec((BLK_B, DIM), lambda i, L: (i, 0)),
                pl.BlockSpec((POOL_SIZE, DIM), lambda i, L: (0, 0)),
            ],
            out_specs=[
                pl.BlockSpec((BLK_B, IDX_PAD), lambda i, L: (i, 0)),
            ],
        ),
        out_shape=[
            jax.ShapeDtypeStruct((B, IDX_PAD), jnp.int32),
        ],
    )(layer_arr, xb, kb)[0]


def _sc_body(table_hbm, idx_hbm, rm_hbm, out_hbm, part_hbm,
             idx_v, rm_v, buf, accv, gsem, wsem, psem):
    wid = lax.axis_index("s") * NC + lax.axis_index("c")
    base = wid * BPW
    pltpu.sync_copy(idx_hbm.at[pl.ds(base, BPW)], idx_v)
    pltpu.sync_copy(rm_hbm.at[pl.ds(0, POOL_SIZE)], rm_v)

    def rm_body(i, acc):
        iv = idx_v[pl.ds(i * 16, 16)]
        pool = lax.rem(iv, jnp.int32(POOL_SIZE))
        return acc + plsc.load_gather(rm_v, [pool])

    acc = lax.fori_loop(0, BPW // 16, rm_body, jnp.zeros((16,), jnp.float32))
    accv[...] = acc
    pltpu.async_copy(accv, part_hbm.at[wid], psem).wait()

    def chunk_body(c, carry):
        pltpu.async_copy(table_hbm.at[idx_v.at[pl.ds(c * CH, CH)]], buf, gsem).wait()
        pltpu.async_copy(buf, out_hbm.at[pl.ds(base + c * CH, CH)], wsem).wait()
        return carry

    lax.fori_loop(0, NCHUNK, chunk_body, 0)


@functools.lru_cache(maxsize=1)
def _make_sc_gather():
    return pl.kernel(
        _sc_body,
        out_type=[
            jax.ShapeDtypeStruct((NIDX, ROW), jnp.float32),
            jax.ShapeDtypeStruct((NW, 16), jnp.float32),
        ],
        mesh=plsc.VectorSubcoreMesh(core_axis_name="c", subcore_axis_name="s"),
        compiler_params=pltpu.CompilerParams(needs_layout_passes=False),
        scratch_types=[
            pltpu.VMEM((BPW,), jnp.int32),
            pltpu.VMEM((POOL_SIZE,), jnp.float32),
            pltpu.VMEM((CH, ROW), jnp.float32),
            pltpu.VMEM((16,), jnp.float32),
            pltpu.SemaphoreType.DMA,
            pltpu.SemaphoreType.DMA,
            pltpu.SemaphoreType.DMA,
        ],
    )


def kernel(x, layer, keys, prompts):
    layer_arr = jnp.asarray(layer, jnp.int32).reshape(1)
    xn, kn, rm = _normalize(layer_arr, x, keys)
    xb = xn.astype(jnp.bfloat16)
    kb = kn.astype(jnp.bfloat16)
    idx_pad = _topk(layer_arr, xb, kb)
    idx_flat = idx_pad[:, :TOPK].reshape(-1)
    table = prompts.reshape(NUM_LAYERS * POOL_SIZE, ROW)
    sel_flat, parts = _make_sc_gather()(table, idx_flat, rm)
    sel = sel_flat.reshape(B, TOPK * LEN_PROMPTS, DIM)
    dist = jnp.float32(1.0) - parts.sum() / jnp.float32(NIDX)
    return (sel, dist)
